# baseline (speedup 1.0000x reference)
import math

import jax
import jax.numpy as jnp
from jax import lax
from jax.experimental import pallas as pl
from jax.experimental.pallas import tpu as pltpu

_F = 256
_P = 4096
_B = 51
_C = 19
_CP = 32
_FB = 256

_BG = 4
_BP = 52
_DN_LAST = (((1,), (1,)), ((), ()))
_DN_STD = (((1,), (0,)), ((), ()))
_LOG2E = 1.4426950408889634


def _hist_kernel(x_ref, lab_ref, out_ref, hist_ref):
    x = x_ref[...]
    lab = lab_ref[...]
    cls = lax.broadcasted_iota(jnp.int32, (_CP, _P), 0)
    onehot = (cls == lab).astype(jnp.float32)

    sums = lax.dot_general(jnp.concatenate([x, x * x], axis=0), onehot,
                           _DN_LAST,
                           preferred_element_type=jnp.float32)
    sum1 = sums[:_FB]
    sum2 = sums[_FB:]
    nrow = lax.dot_general(jnp.ones((1, _P), jnp.float32), onehot, _DN_LAST,
                           preferred_element_type=jnp.float32)

    inv_n = 1.0 / jnp.maximum(nrow, 1.0)
    miu = sum1 * inv_n
    var = jnp.maximum(sum2 * inv_n - miu * miu, 1e-12)
    inv2var = (0.5 * _LOG2E) / var
    neg_inv2vs = (-12.5 * _LOG2E) / var

    a = lax.dot_general(neg_inv2vs, onehot, _DN_STD,
                        preferred_element_type=jnp.float32)

    boff = (lax.broadcasted_iota(jnp.int32, (_BG, 1, 1), 0)
            .astype(jnp.float32) * 0.2)
    x3 = x.astype(jnp.bfloat16)[None]
    a3 = a.astype(jnp.bfloat16)[None]
    onehot_b = onehot.astype(jnp.bfloat16)

    def kde_body(i, carry):
        bks = ((-5.0 + (0.2 * _BG) * i.astype(jnp.float32)) + boff
               ).astype(jnp.bfloat16)
        d = bks - x3
        e = jnp.exp2(d * d * a3).reshape(_BG * _FB, _P)
        h = lax.dot_general(e, onehot_b, _DN_LAST,
                            preferred_element_type=jnp.float32)
        hist_ref[pl.ds(i * _BG, _BG)] = h.reshape(_BG, _FB, _CP)
        return carry
    lax.fori_loop(0, _BP // _BG, kde_body, 0, unroll=2)

    def sum_body(k, carry):
        denom, tsum = carry
        bk = -5.0 + 0.2 * k.astype(jnp.float32)
        dt = bk - miu
        return denom + hist_ref[k], tsum + jnp.exp2(-(dt * dt) * inv2var)
    denom, tsum = lax.fori_loop(
        0, _B, sum_body,
        (jnp.zeros((_FB, _CP), jnp.float32), jnp.zeros((_FB, _CP), jnp.float32)))

    s = lax.rsqrt((2.0 * math.pi / 25.0) * var)
    factor = s / jnp.maximum(denom * s, 1e-12)
    inv_tsum = 1.0 / tsum

    def sl1_body(k, acc):
        bk = -5.0 + 0.2 * k.astype(jnp.float32)
        dt = bk - miu
        tgt = jnp.exp2(-(dt * dt) * inv2var) * inv_tsum
        diff = hist_ref[k] * factor - tgt
        ad = jnp.abs(diff)
        return acc + jnp.where(ad < 1.0, 0.5 * diff * diff, ad - 0.5)
    sl1 = lax.fori_loop(0, _B, sl1_body, jnp.zeros((_FB, _CP), jnp.float32))

    out_ref[...] = jnp.concatenate(
        [jnp.sum(sl1, axis=0, keepdims=True) * (1.0 / (_F * _B)), nrow],
        axis=0)


def _run(x, lab):
    return pl.pallas_call(
        _hist_kernel,
        out_shape=jax.ShapeDtypeStruct((2, _CP), jnp.float32),
        scratch_shapes=[pltpu.VMEM((_BP, _FB, _CP), jnp.float32)],
        name="histogram_loss",
    )(x, lab)


def kernel(feature, label):
    x = feature[0].reshape(_F, _P)
    lab = label.reshape(1, _P).astype(jnp.int32)
    parts = _run(x, lab)
    sl1_c = parts[0, :_C]
    n_c = parts[1, 1:_C]
    has = (n_c > 0).astype(jnp.float32)
    loss = jnp.sum(sl1_c[1:] * has)
    return loss / (jnp.sum(has) + 1e-12)


# speedup vs baseline: 1.0522x; 1.0522x over previous
import math

import jax
import jax.numpy as jnp
from jax import lax
from jax.experimental import pallas as pl
from jax.experimental.pallas import tpu as pltpu

_F = 256
_P = 4096
_B = 51
_C = 19
_CP = 32
_FB = 256

_BG = 8
_BP = 56
_DN_LAST = (((1,), (1,)), ((), ()))
_DN_STD = (((1,), (0,)), ((), ()))
_LOG2E = 1.4426950408889634


def _hist_kernel(x_ref, lab_ref, out_ref, hist_ref):
    x = x_ref[...]
    lab = lab_ref[...]
    cls = lax.broadcasted_iota(jnp.int32, (_CP, _P), 0)
    onehot = (cls == lab).astype(jnp.float32)

    sums = lax.dot_general(jnp.concatenate([x, x * x], axis=0), onehot,
                           _DN_LAST,
                           preferred_element_type=jnp.float32)
    sum1 = sums[:_FB]
    sum2 = sums[_FB:]
    nrow = lax.dot_general(jnp.ones((1, _P), jnp.float32), onehot, _DN_LAST,
                           preferred_element_type=jnp.float32)

    inv_n = 1.0 / jnp.maximum(nrow, 1.0)
    miu = sum1 * inv_n
    var = jnp.maximum(sum2 * inv_n - miu * miu, 1e-12)
    inv2var = (0.5 * _LOG2E) / var
    neg_inv2vs = (-12.5 * _LOG2E) / var

    a = lax.dot_general(neg_inv2vs, onehot, _DN_STD,
                        preferred_element_type=jnp.float32)

    boff = (lax.broadcasted_iota(jnp.int32, (_BG, 1, 1), 0)
            .astype(jnp.float32) * 0.2)
    x3 = x.astype(jnp.bfloat16)[None]
    a3 = a.astype(jnp.bfloat16)[None]
    onehot_b = onehot.astype(jnp.bfloat16)

    def kde_body(i, carry):
        bks = ((-5.0 + (0.2 * _BG) * i.astype(jnp.float32)) + boff
               ).astype(jnp.bfloat16)
        d = bks - x3
        e = jnp.exp2(d * d * a3).reshape(_BG * _FB, _P)
        h = lax.dot_general(e, onehot_b, _DN_LAST,
                            preferred_element_type=jnp.float32)
        hist_ref[pl.ds(i * _BG, _BG)] = h.reshape(_BG, _FB, _CP)
        return carry
    lax.fori_loop(0, _BP // _BG, kde_body, 0, unroll=2)

    kb3 = (lax.broadcasted_iota(jnp.int32, (_B, 1, 1), 0)
           .astype(jnp.float32) * 0.2 - 5.0)
    dt3 = kb3 - miu[None]
    tpdf = jnp.exp2(-(dt3 * dt3) * inv2var[None])
    tsum = jnp.sum(tpdf, axis=0)
    hist3 = hist_ref[: _B]
    denom = jnp.sum(hist3, axis=0)

    s = lax.rsqrt((2.0 * math.pi / 25.0) * var)
    factor = s / jnp.maximum(denom * s, 1e-12)
    diff = hist3 * factor[None] - tpdf * (1.0 / tsum)[None]
    ad = jnp.abs(diff)
    sl13 = jnp.where(ad < 1.0, 0.5 * diff * diff, ad - 0.5)
    colsum = jnp.sum(jnp.sum(sl13, axis=0), axis=0,
                     keepdims=True) * (1.0 / (_F * _B))

    crow = lax.broadcasted_iota(jnp.int32, (1, _CP), 1)
    act = jnp.where((crow >= 1) & (crow < _C) & (nrow > 0.0), 1.0, 0.0)
    loss = jnp.sum(colsum * act, axis=1, keepdims=True)
    out_ref[...] = loss / (jnp.sum(act, axis=1, keepdims=True) + 1e-12)


def _run(x, lab):
    return pl.pallas_call(
        _hist_kernel,
        out_shape=jax.ShapeDtypeStruct((1, 1), jnp.float32),
        scratch_shapes=[pltpu.VMEM((_BP, _FB, _CP), jnp.float32)],
        name="histogram_loss",
    )(x, lab)


def kernel(feature, label):
    x = feature[0].reshape(_F, _P)
    lab = label.reshape(1, _P).astype(jnp.int32)
    return _run(x, lab).reshape(())


# speedup vs baseline: 1.0612x; 1.0086x over previous
import math

import jax
import jax.numpy as jnp
from jax import lax
from jax.experimental import pallas as pl
from jax.experimental.pallas import tpu as pltpu

_F = 256
_P = 4096
_B = 51
_C = 19
_CP = 32
_FB = 256

_BG = 8
_BP = 56
_DN_LAST = (((1,), (1,)), ((), ()))
_DN_STD = (((1,), (0,)), ((), ()))
_LOG2E = 1.4426950408889634


def _hist_kernel(x_ref, lab_ref, out_ref, hist_ref):
    x = x_ref[...]
    lab = lab_ref[...]
    cls = lax.broadcasted_iota(jnp.int32, (_CP, _P), 0)
    onehot = (cls == lab).astype(jnp.float32)

    sums = lax.dot_general(jnp.concatenate([x, x * x], axis=0), onehot,
                           _DN_LAST,
                           preferred_element_type=jnp.float32)
    sum1 = sums[:_FB]
    sum2 = sums[_FB:]
    nrow = lax.dot_general(jnp.ones((1, _P), jnp.float32), onehot, _DN_LAST,
                           preferred_element_type=jnp.float32)

    inv_n = 1.0 / jnp.maximum(nrow, 1.0)
    miu = sum1 * inv_n
    var = jnp.maximum(sum2 * inv_n - miu * miu, 1e-12)
    inv2var = (0.5 * _LOG2E) / var
    neg_inv2vs = (-12.5 * _LOG2E) / var

    a = lax.dot_general(neg_inv2vs, onehot, _DN_STD,
                        preferred_element_type=jnp.float32)

    boff = (lax.broadcasted_iota(jnp.int32, (_BG, 1, 1), 0)
            .astype(jnp.float32) * 0.2)
    x3 = x.astype(jnp.bfloat16)[None]
    a3 = a.astype(jnp.bfloat16)[None]
    onehot_b = onehot.astype(jnp.bfloat16)

    for i in range(_BP // _BG):
        bks = ((-5.0 + (0.2 * _BG) * float(i)) + boff).astype(jnp.bfloat16)
        d = bks - x3
        e = jnp.exp2(d * d * a3).reshape(_BG * _FB, _P)
        h = lax.dot_general(e, onehot_b, _DN_LAST,
                            preferred_element_type=jnp.float32)
        hist_ref[i * _BG:(i + 1) * _BG] = h.reshape(_BG, _FB, _CP)

    kb3 = (lax.broadcasted_iota(jnp.int32, (_B, 1, 1), 0)
           .astype(jnp.float32) * 0.2 - 5.0)
    dt3 = kb3 - miu[None]
    tpdf = jnp.exp2(-(dt3 * dt3) * inv2var[None])
    tsum = jnp.sum(tpdf, axis=0)
    hist3 = hist_ref[: _B]
    denom = jnp.sum(hist3, axis=0)

    s = lax.rsqrt((2.0 * math.pi / 25.0) * var)
    factor = s / jnp.maximum(denom * s, 1e-12)
    diff = hist3 * factor[None] - tpdf * (1.0 / tsum)[None]
    ad = jnp.abs(diff)
    sl13 = jnp.where(ad < 1.0, 0.5 * diff * diff, ad - 0.5)
    colsum = jnp.sum(jnp.sum(sl13, axis=0), axis=0,
                     keepdims=True) * (1.0 / (_F * _B))

    crow = lax.broadcasted_iota(jnp.int32, (1, _CP), 1)
    act = jnp.where((crow >= 1) & (crow < _C) & (nrow > 0.0), 1.0, 0.0)
    loss = jnp.sum(colsum * act, axis=1, keepdims=True)
    out_ref[...] = loss / (jnp.sum(act, axis=1, keepdims=True) + 1e-12)


def _run(x, lab):
    return pl.pallas_call(
        _hist_kernel,
        out_shape=jax.ShapeDtypeStruct((1, 1), jnp.float32),
        scratch_shapes=[pltpu.VMEM((_BP, _FB, _CP), jnp.float32)],
        name="histogram_loss",
    )(x, lab)


def kernel(feature, label):
    x = feature[0].reshape(_F, _P)
    lab = label.reshape(1, _P).astype(jnp.int32)
    return _run(x, lab).reshape(())


# speedup vs baseline: 1.0783x; 1.0161x over previous
import math

import jax
import jax.numpy as jnp
from jax import lax
from jax.experimental import pallas as pl
from jax.experimental.pallas import tpu as pltpu

_F = 256
_P = 4096
_B = 51
_C = 19
_CP = 32
_FB = 256

_BG = 8
_BP = 56
_DN_LAST = (((1,), (1,)), ((), ()))
_DN_STD = (((1,), (0,)), ((), ()))
_LOG2E = 1.4426950408889634


def _hist_kernel(x_ref, lab_ref, out_ref, hist_ref):
    x = x_ref[...]
    lab = lab_ref[...]
    cls = lax.broadcasted_iota(jnp.int32, (_CP, _P), 0)
    onehot = (cls == lab).astype(jnp.float32)

    sums = lax.dot_general(jnp.concatenate([x, x * x], axis=0), onehot,
                           _DN_LAST,
                           preferred_element_type=jnp.float32)
    sum1 = sums[:_FB]
    sum2 = sums[_FB:]
    nrow = lax.dot_general(jnp.ones((1, _P), jnp.float32), onehot, _DN_LAST,
                           preferred_element_type=jnp.float32)

    inv_n = 1.0 / jnp.maximum(nrow, 1.0)
    miu = sum1 * inv_n
    var = jnp.maximum(sum2 * inv_n - miu * miu, 1e-12)
    inv2var = (0.5 * _LOG2E) / var
    neg_inv2vs = (-12.5 * _LOG2E) / var

    a = lax.dot_general(neg_inv2vs, onehot, _DN_STD,
                        preferred_element_type=jnp.float32)

    boff = (lax.broadcasted_iota(jnp.int32, (_BG, 1, 1), 0)
            .astype(jnp.float32) * 0.2)
    x3 = x.astype(jnp.bfloat16)[None]
    a3 = a.astype(jnp.bfloat16)[None]
    onehot_b = onehot.astype(jnp.bfloat16)

    _RC = 64
    for i in range(_BP // _BG):
        bks = ((-5.0 + (0.2 * _BG) * float(i)) + boff).astype(jnp.bfloat16)
        hs = []
        for rc in range(_FB // _RC):
            xc = x3[:, rc * _RC:(rc + 1) * _RC, :]
            ac = a3[:, rc * _RC:(rc + 1) * _RC, :]
            dd = bks - xc
            ec = jnp.exp2(dd * dd * ac).reshape(_BG * _RC, _P)
            hs.append(lax.dot_general(ec, onehot_b, _DN_LAST,
                                      preferred_element_type=jnp.float32)
                      .reshape(_BG, _RC, _CP))
        hist_ref[i * _BG:(i + 1) * _BG] = jnp.concatenate(hs, axis=1)

    kb3 = (lax.broadcasted_iota(jnp.int32, (_B, 1, 1), 0)
           .astype(jnp.float32) * 0.2 - 5.0)
    dt3 = kb3 - miu[None]
    tpdf = jnp.exp2(-(dt3 * dt3) * inv2var[None])
    tsum = jnp.sum(tpdf, axis=0)
    hist3 = hist_ref[: _B]
    denom = jnp.sum(hist3, axis=0)

    s = lax.rsqrt((2.0 * math.pi / 25.0) * var)
    factor = s / jnp.maximum(denom * s, 1e-12)
    diff = hist3 * factor[None] - tpdf * (1.0 / tsum)[None]
    ad = jnp.abs(diff)
    sl13 = jnp.where(ad < 1.0, 0.5 * diff * diff, ad - 0.5)
    colsum = jnp.sum(jnp.sum(sl13, axis=0), axis=0,
                     keepdims=True) * (1.0 / (_F * _B))

    crow = lax.broadcasted_iota(jnp.int32, (1, _CP), 1)
    act = jnp.where((crow >= 1) & (crow < _C) & (nrow > 0.0), 1.0, 0.0)
    loss = jnp.sum(colsum * act, axis=1, keepdims=True)
    out_ref[...] = loss / (jnp.sum(act, axis=1, keepdims=True) + 1e-12)


def _run(x, lab):
    return pl.pallas_call(
        _hist_kernel,
        out_shape=jax.ShapeDtypeStruct((1, 1), jnp.float32),
        scratch_shapes=[pltpu.VMEM((_BP, _FB, _CP), jnp.float32)],
        name="histogram_loss",
    )(x, lab)


def kernel(feature, label):
    x = feature[0].reshape(_F, _P)
    lab = label.reshape(1, _P).astype(jnp.int32)
    return _run(x, lab).reshape(())
